# Initial kernel scaffold; baseline (speedup 1.0000x reference)
import jax
import jax.numpy as jnp
from jax.experimental import pallas as pl


def kernel(x, table):
    raise NotImplementedError("write your pallas kernel here")


# speedup vs baseline: 1.2983x; 1.2983x over previous
import functools

import jax
import jax.numpy as jnp
from jax import lax
from jax.experimental import pallas as pl
from jax.experimental.pallas import tpu as pltpu
from jax.experimental.pallas import tpu_sc as plsc

_INFO = plsc.get_sparse_core_info()
_NC, _NS = _INFO.num_cores, _INFO.num_subcores
_NW = _NC * _NS

_CHUNK = 128
_LANE = 128


def _make_gather(vocab: int, dim: int, dim_pad: int, n_idx: int):
  assert n_idx % (_NW * _CHUNK) == 0
  per_w = n_idx // _NW
  n_chunks = per_w // _CHUNK
  mesh = plsc.VectorSubcoreMesh(core_axis_name="c", subcore_axis_name="s")

  @functools.partial(
      pl.kernel,
      mesh=mesh,
      out_type=jax.ShapeDtypeStruct((n_idx, dim_pad), jnp.float32),
      scratch_types=[
          pltpu.VMEM((n_chunks, _CHUNK), jnp.int32),
          pltpu.VMEM((_CHUNK, dim_pad), jnp.float32),
          pltpu.SemaphoreType.DMA,
      ],
  )
  def gather_kernel(table_hbm, idx_hbm, out_hbm, idx_v, rows_v, sem):
    wid = lax.axis_index("s") * _NC + lax.axis_index("c")
    pltpu.sync_copy(idx_hbm.at[pl.ds(wid * n_chunks, n_chunks)], idx_v)
    base = wid * per_w

    def chunk_body(c, carry):
      pltpu.async_copy(table_hbm.at[idx_v.at[c]], rows_v, sem).wait()
      pltpu.sync_copy(rows_v, out_hbm.at[pl.ds(base + c * _CHUNK, _CHUNK)])
      return carry

    lax.fori_loop(0, n_chunks, chunk_body, 0)

  return gather_kernel


def kernel(x, table):
  batch, hist = x.shape
  vocab, dim = table.shape
  dim_pad = (dim + _LANE - 1) // _LANE * _LANE
  n_idx = batch * hist
  idx2d = x.reshape(n_idx // _CHUNK, _CHUNK).astype(jnp.int32)
  table_p = jnp.pad(table, ((0, 0), (0, dim_pad - dim)))
  out = _make_gather(vocab, dim, dim_pad, n_idx)(table_p, idx2d)
  return out[:, :dim].reshape(batch, hist, dim)
